# baseline (speedup 1.0000x reference)
import jax
import jax.numpy as jnp
from jax import lax
from jax.experimental import pallas as pl
from jax.experimental.pallas import tpu as pltpu
from jax.experimental.pallas import tpu_sc as plsc

BATCH = 16384
HID = 128

_info = plsc.get_sparse_core_info()
_NC, _NS, _L = _info.num_cores, _info.num_subcores, _info.num_lanes
_NW = _NC * _NS
_BPW = BATCH // _NW
_CHUNK = 128
_NCHUNK = _BPW // _CHUNK
_GROUPS = _CHUNK // _L
_DSLICES = HID // _L


def _dot_chunk(u_buf, m_buf, out_v, out_base, iota):
    def group_body(g, carry):
        row0 = g * _L
        out16 = jnp.zeros((_L,), jnp.float32)
        for r in range(_L):
            row = row0 + r
            acc = (u_buf[row, pl.ds(0, _L)] * m_buf[row, pl.ds(0, _L)])
            for d in range(1, _DSLICES):
                acc = acc + (u_buf[row, pl.ds(d * _L, _L)]
                             * m_buf[row, pl.ds(d * _L, _L)])
            out16 = jnp.where(iota == r, jnp.sum(acc), out16)
        out_v[pl.ds(out_base + row0, _L)] = out16
        return carry

    lax.fori_loop(0, _GROUPS, group_body, 0)


def _body(ui_hbm, ut_hbm, mt_hbm, out_hbm,
          ui_v, uidx, midx, u0, u1, m0, m1, out_v, sem0, sem1):
    wid = lax.axis_index("s") * _NC + lax.axis_index("c")
    base = wid * _BPW
    iota = lax.iota(jnp.int32, _L)

    pltpu.sync_copy(ui_hbm.at[pl.ds(base * 2, _BPW * 2)], ui_v)

    for j in range(_BPW // _L):
        flat = (iota + (j * _L)) * 2
        c, s = j // (_CHUNK // _L), (j % (_CHUNK // _L)) * _L
        uidx[c, pl.ds(s, _L)] = plsc.load_gather(ui_v, [flat])
        midx[c, pl.ds(s, _L)] = plsc.load_gather(ui_v, [flat + 1])

    ubufs, mbufs, sems = (u0, u1), (m0, m1), (sem0, sem1)

    def issue(c):
        b = c % 2
        hu = pltpu.async_copy(ut_hbm.at[uidx.at[c]], ubufs[b], sems[b])
        hm = pltpu.async_copy(mt_hbm.at[midx.at[c]], mbufs[b], sems[b])
        return hu, hm

    handles = {0: issue(0), 1: issue(1)}
    for c in range(_NCHUNK):
        hu, hm = handles[c]
        hu.wait()
        hm.wait()
        b = c % 2
        _dot_chunk(ubufs[b], mbufs[b], out_v, c * _CHUNK, iota)
        if c + 2 < _NCHUNK:
            handles[c + 2] = issue(c + 2)

    pltpu.sync_copy(out_v, out_hbm.at[pl.ds(base, _BPW)])


def kernel(ui, user_table, movie_table):
    ui = ui.astype(jnp.int32).reshape(-1)
    mesh = plsc.VectorSubcoreMesh(core_axis_name="c", subcore_axis_name="s")
    f = pl.kernel(
        _body,
        mesh=mesh,
        out_type=jax.ShapeDtypeStruct((BATCH,), jnp.float32),
        compiler_params=pltpu.CompilerParams(needs_layout_passes=False),
        scratch_types=[
            pltpu.VMEM((_BPW * 2,), jnp.int32),
            pltpu.VMEM((_NCHUNK, _CHUNK), jnp.int32),
            pltpu.VMEM((_NCHUNK, _CHUNK), jnp.int32),
            pltpu.VMEM((_CHUNK, HID), jnp.float32),
            pltpu.VMEM((_CHUNK, HID), jnp.float32),
            pltpu.VMEM((_CHUNK, HID), jnp.float32),
            pltpu.VMEM((_CHUNK, HID), jnp.float32),
            pltpu.VMEM((_BPW,), jnp.float32),
            pltpu.SemaphoreType.DMA,
            pltpu.SemaphoreType.DMA,
        ],
    )
    return f(ui, user_table, movie_table)
